# baseline (speedup 1.0000x reference)
import functools

import jax
import jax.numpy as jnp
from jax import lax
from jax.experimental import pallas as pl
from jax.experimental.pallas import tpu as pltpu
from jax.experimental.pallas import tpu_sc as plsc

ENTITY_NUM = 100000
HIDDEN_DIM = 128
E = ENTITY_NUM * HIDDEN_DIM

NC = 2
NS = 16
NW = NC * NS
LANES = 16

PER_W = E // NW
DEPTH = 4
CHUNK = 4000
NCHUNK = PER_W // CHUNK

_mesh = plsc.VectorSubcoreMesh(core_axis_name="c", subcore_axis_name="s")


@functools.partial(
    pl.kernel,
    mesh=_mesh,
    out_type=[
        jax.ShapeDtypeStruct((E,), jnp.float32),
        jax.ShapeDtypeStruct((E,), jnp.float32),
    ],
    scratch_types=(
        [pltpu.VMEM((CHUNK,), jnp.float32)] * (5 * DEPTH)
        + [pltpu.SemaphoreType.DMA] * (2 * DEPTH)
    ),
)
def _gate_fuse(x_hbm, y_hbm, t_hbm, out_hbm, gate_hbm, *scratch):
    bufs = scratch[: 5 * DEPTH]
    sems = scratch[5 * DEPTH:]
    xv = bufs[0:DEPTH]
    yv = bufs[DEPTH:2 * DEPTH]
    tv = bufs[2 * DEPTH:3 * DEPTH]
    ov = bufs[3 * DEPTH:4 * DEPTH]
    gv = bufs[4 * DEPTH:5 * DEPTH]
    sem_in = sems[0:DEPTH]
    sem_out = sems[DEPTH:2 * DEPTH]

    wid = lax.axis_index("s") * NC + lax.axis_index("c")
    base = wid * PER_W

    def start_in(c, b):
        off = base + c * CHUNK
        pltpu.async_copy(x_hbm.at[pl.ds(off, CHUNK)], xv[b], sem_in[b])
        pltpu.async_copy(y_hbm.at[pl.ds(off, CHUNK)], yv[b], sem_in[b])
        pltpu.async_copy(t_hbm.at[pl.ds(off, CHUNK)], tv[b], sem_in[b])

    def drain_in(b):
        for dst in (xv[b], yv[b], tv[b]):
            pltpu.make_async_copy(x_hbm.at[pl.ds(0, CHUNK)], dst, sem_in[b]).wait()

    def start_out(c, b):
        off = base + c * CHUNK
        pltpu.async_copy(ov[b], out_hbm.at[pl.ds(off, CHUNK)], sem_out[b])
        pltpu.async_copy(gv[b], gate_hbm.at[pl.ds(off, CHUNK)], sem_out[b])

    def drain_out(b):
        pltpu.make_async_copy(ov[b], out_hbm.at[pl.ds(0, CHUNK)], sem_out[b]).wait()
        pltpu.make_async_copy(gv[b], gate_hbm.at[pl.ds(0, CHUNK)], sem_out[b]).wait()

    for b in range(DEPTH):
        start_in(b, b)

    def round_body(g, carry):
        for b in range(DEPTH):
            c = DEPTH * g + b
            drain_in(b)

            @pl.when(g > 0)
            def _():
                drain_out(b)

            @plsc.parallel_loop(0, CHUNK, step=LANES, unroll=16)
            def _(i):
                s = pl.ds(i, LANES)
                t = tv[b][s]
                g16 = 1.0 / (1.0 + jnp.exp(-t))
                gv[b][s] = g16
                ov[b][s] = yv[b][s] + g16 * (xv[b][s] - yv[b][s])

            start_out(c, b)

            @pl.when(c + DEPTH < NCHUNK)
            def _():
                start_in(c + DEPTH, b)

        return carry

    lax.fori_loop(0, NCHUNK // DEPTH, round_body, 0)
    for b in range(DEPTH):
        drain_out(b)


def kernel(X, Y, gate_theta):
    x = X.reshape(E)
    y = Y.reshape(E)
    t = gate_theta.reshape(E)
    out, gate = _gate_fuse(x, y, t)
    return out.reshape(X.shape), gate.reshape(X.shape)


# speedup vs baseline: 1.0009x; 1.0009x over previous
import functools

import jax
import jax.numpy as jnp
from jax import lax
from jax.experimental import pallas as pl
from jax.experimental.pallas import tpu as pltpu
from jax.experimental.pallas import tpu_sc as plsc

ENTITY_NUM = 100000
HIDDEN_DIM = 128
E = ENTITY_NUM * HIDDEN_DIM

NC = 2
NS = 16
NW = NC * NS
LANES = 16

PER_W = E // NW
DEPTH = 2
CHUNK = 10000
HALF = CHUNK // 2
NCHUNK = PER_W // CHUNK

_mesh = plsc.VectorSubcoreMesh(core_axis_name="c", subcore_axis_name="s")


@functools.partial(
    pl.kernel,
    mesh=_mesh,
    out_type=[
        jax.ShapeDtypeStruct((E,), jnp.float32),
        jax.ShapeDtypeStruct((E,), jnp.float32),
    ],
    scratch_types=(
        [pltpu.VMEM((CHUNK,), jnp.float32)] * (5 * DEPTH)
        + [pltpu.SemaphoreType.DMA] * (2 * DEPTH)
    ),
)
def _gate_fuse(x_hbm, y_hbm, t_hbm, out_hbm, gate_hbm, *scratch):
    bufs = scratch[: 5 * DEPTH]
    sems = scratch[5 * DEPTH:]
    xv = bufs[0:DEPTH]
    yv = bufs[DEPTH:2 * DEPTH]
    tv = bufs[2 * DEPTH:3 * DEPTH]
    ov = bufs[3 * DEPTH:4 * DEPTH]
    gv = bufs[4 * DEPTH:5 * DEPTH]
    sem_in = sems[0:DEPTH]
    sem_out = sems[DEPTH:2 * DEPTH]

    wid = lax.axis_index("s") * NC + lax.axis_index("c")
    base = wid * PER_W

    def start_in(c, b):
        off = base + c * CHUNK
        for src, dst in ((x_hbm, xv[b]), (y_hbm, yv[b]), (t_hbm, tv[b])):
            pltpu.async_copy(src.at[pl.ds(off, HALF)], dst.at[pl.ds(0, HALF)],
                             sem_in[b])
            pltpu.async_copy(src.at[pl.ds(off + HALF, HALF)],
                             dst.at[pl.ds(HALF, HALF)], sem_in[b])

    def drain_in(b):
        for dst in (xv[b], yv[b], tv[b]):
            pltpu.make_async_copy(x_hbm.at[pl.ds(0, CHUNK)], dst, sem_in[b]).wait()

    def start_out(c, b):
        off = base + c * CHUNK
        for src, dst in ((ov[b], out_hbm), (gv[b], gate_hbm)):
            pltpu.async_copy(src.at[pl.ds(0, HALF)], dst.at[pl.ds(off, HALF)],
                             sem_out[b])
            pltpu.async_copy(src.at[pl.ds(HALF, HALF)],
                             dst.at[pl.ds(off + HALF, HALF)], sem_out[b])

    def drain_out(b):
        pltpu.make_async_copy(ov[b], out_hbm.at[pl.ds(0, CHUNK)], sem_out[b]).wait()
        pltpu.make_async_copy(gv[b], gate_hbm.at[pl.ds(0, CHUNK)], sem_out[b]).wait()

    for b in range(DEPTH):
        start_in(b, b)

    def round_body(g, carry):
        for b in range(DEPTH):
            c = DEPTH * g + b
            drain_in(b)

            @pl.when(g > 0)
            def _():
                drain_out(b)

            @plsc.parallel_loop(0, CHUNK, step=LANES, unroll=16)
            def _(i):
                s = pl.ds(i, LANES)
                t = tv[b][s]
                g16 = 1.0 / (1.0 + jnp.exp(-t))
                gv[b][s] = g16
                ov[b][s] = yv[b][s] + g16 * (xv[b][s] - yv[b][s])

            start_out(c, b)

            @pl.when(c + DEPTH < NCHUNK)
            def _():
                start_in(c + DEPTH, b)

        return carry

    lax.fori_loop(0, NCHUNK // DEPTH, round_body, 0)
    for b in range(DEPTH):
        drain_out(b)


def kernel(X, Y, gate_theta):
    x = X.reshape(E)
    y = Y.reshape(E)
    t = gate_theta.reshape(E)
    out, gate = _gate_fuse(x, y, t)
    return out.reshape(X.shape), gate.reshape(X.shape)


# speedup vs baseline: 1.0076x; 1.0067x over previous
import functools

import jax
import jax.numpy as jnp
from jax import lax
from jax.experimental import pallas as pl
from jax.experimental.pallas import tpu as pltpu
from jax.experimental.pallas import tpu_sc as plsc

ENTITY_NUM = 100000
HIDDEN_DIM = 128
E = ENTITY_NUM * HIDDEN_DIM

NC = 2
NS = 16
NW = NC * NS
LANES = 16

PER_W = E // NW
DEPTH = 2
CHUNK = 10000
NCHUNK = PER_W // CHUNK

_mesh = plsc.VectorSubcoreMesh(core_axis_name="c", subcore_axis_name="s")


@functools.partial(
    pl.kernel,
    mesh=_mesh,
    out_type=[
        jax.ShapeDtypeStruct((E,), jnp.float32),
        jax.ShapeDtypeStruct((E,), jnp.float32),
    ],
    scratch_types=(
        [pltpu.VMEM((CHUNK,), jnp.float32)] * (5 * DEPTH)
        + [pltpu.SemaphoreType.DMA] * (2 * DEPTH)
    ),
)
def _gate_fuse(x_hbm, y_hbm, t_hbm, out_hbm, gate_hbm, *scratch):
    bufs = scratch[: 5 * DEPTH]
    sems = scratch[5 * DEPTH:]
    xv = bufs[0:DEPTH]
    yv = bufs[DEPTH:2 * DEPTH]
    tv = bufs[2 * DEPTH:3 * DEPTH]
    ov = bufs[3 * DEPTH:4 * DEPTH]
    gv = bufs[4 * DEPTH:5 * DEPTH]
    sem_in = sems[0:DEPTH]
    sem_out = sems[DEPTH:2 * DEPTH]

    wid = lax.axis_index("s") * NC + lax.axis_index("c")
    base = wid * PER_W

    def start_in(c, b):
        off = base + c * CHUNK
        pltpu.async_copy(x_hbm.at[pl.ds(off, CHUNK)], xv[b], sem_in[b])
        pltpu.async_copy(y_hbm.at[pl.ds(off, CHUNK)], yv[b], sem_in[b])
        pltpu.async_copy(t_hbm.at[pl.ds(off, CHUNK)], tv[b], sem_in[b])

    def drain_in(b):
        for dst in (xv[b], yv[b], tv[b]):
            pltpu.make_async_copy(x_hbm.at[pl.ds(0, CHUNK)], dst, sem_in[b]).wait()

    def start_out(c, b):
        off = base + c * CHUNK
        pltpu.async_copy(ov[b], out_hbm.at[pl.ds(off, CHUNK)], sem_out[b])
        pltpu.async_copy(gv[b], gate_hbm.at[pl.ds(off, CHUNK)], sem_out[b])

    def drain_out(b):
        pltpu.make_async_copy(ov[b], out_hbm.at[pl.ds(0, CHUNK)], sem_out[b]).wait()
        pltpu.make_async_copy(gv[b], gate_hbm.at[pl.ds(0, CHUNK)], sem_out[b]).wait()

    for b in range(DEPTH):
        start_in(b, b)

    def round_body(g, carry):
        for b in range(DEPTH):
            c = DEPTH * g + b
            drain_in(b)

            @pl.when(g > 0)
            def _():
                drain_out(b)

            @plsc.parallel_loop(0, CHUNK, step=LANES, unroll=16)
            def _(i):
                s = pl.ds(i, LANES)
                t = tv[b][s]
                g16 = 1.0 / (1.0 + jnp.exp(-t))
                gv[b][s] = g16
                ov[b][s] = yv[b][s] + g16 * (xv[b][s] - yv[b][s])

            start_out(c, b)

            @pl.when(c + DEPTH < NCHUNK)
            def _():
                start_in(c + DEPTH, b)

        return carry

    lax.fori_loop(0, NCHUNK // DEPTH, round_body, 0)
    for b in range(DEPTH):
        drain_out(b)


def kernel(X, Y, gate_theta):
    x = X.reshape(E)
    y = Y.reshape(E)
    t = gate_theta.reshape(E)
    out, gate = _gate_fuse(x, y, t)
    return out.reshape(X.shape), gate.reshape(X.shape)


# speedup vs baseline: 1.0081x; 1.0005x over previous
import functools

import jax
import jax.numpy as jnp
from jax import lax
from jax.experimental import pallas as pl
from jax.experimental.pallas import tpu as pltpu
from jax.experimental.pallas import tpu_sc as plsc

ENTITY_NUM = 100000
HIDDEN_DIM = 128
E = ENTITY_NUM * HIDDEN_DIM

NC = 2
NS = 16
NW = NC * NS
LANES = 16

PER_W = E // NW
DEPTH = 2
CHUNK = 10000
NCHUNK = PER_W // CHUNK

_mesh = plsc.VectorSubcoreMesh(core_axis_name="c", subcore_axis_name="s")


@functools.partial(
    pl.kernel,
    mesh=_mesh,
    out_type=[
        jax.ShapeDtypeStruct((E,), jnp.float32),
        jax.ShapeDtypeStruct((E,), jnp.float32),
    ],
    scratch_types=(
        [pltpu.VMEM((CHUNK,), jnp.float32)] * (5 * DEPTH)
        + [pltpu.SemaphoreType.DMA] * (2 * DEPTH)
    ),
)
def _gate_fuse(x_hbm, y_hbm, t_hbm, out_hbm, gate_hbm, *scratch):
    bufs = scratch[: 5 * DEPTH]
    sems = scratch[5 * DEPTH:]
    xv = bufs[0:DEPTH]
    yv = bufs[DEPTH:2 * DEPTH]
    tv = bufs[2 * DEPTH:3 * DEPTH]
    ov = bufs[3 * DEPTH:4 * DEPTH]
    gv = bufs[4 * DEPTH:5 * DEPTH]
    sem_in = sems[0:DEPTH]
    sem_out = sems[DEPTH:2 * DEPTH]

    wid = lax.axis_index("s") * NC + lax.axis_index("c")
    base = wid * PER_W

    def start_in(c, b):
        off = base + c * CHUNK
        pltpu.async_copy(x_hbm.at[pl.ds(off, CHUNK)], xv[b], sem_in[b])
        pltpu.async_copy(y_hbm.at[pl.ds(off, CHUNK)], yv[b], sem_in[b])
        pltpu.async_copy(t_hbm.at[pl.ds(off, CHUNK)], tv[b], sem_in[b])

    def drain_in(b):
        for dst in (xv[b], yv[b], tv[b]):
            pltpu.make_async_copy(x_hbm.at[pl.ds(0, CHUNK)], dst, sem_in[b]).wait()

    def start_out(c, b):
        off = base + c * CHUNK
        pltpu.async_copy(ov[b], out_hbm.at[pl.ds(off, CHUNK)], sem_out[b])
        pltpu.async_copy(gv[b], gate_hbm.at[pl.ds(off, CHUNK)], sem_out[b])

    def drain_out(b):
        pltpu.make_async_copy(ov[b], out_hbm.at[pl.ds(0, CHUNK)], sem_out[b]).wait()
        pltpu.make_async_copy(gv[b], gate_hbm.at[pl.ds(0, CHUNK)], sem_out[b]).wait()

    for b in range(DEPTH):
        start_in(b, b)

    def round_body(g, carry):
        for b in range(DEPTH):
            c = DEPTH * g + b
            drain_in(b)

            @pl.when(g > 0)
            def _():
                drain_out(b)

            @plsc.parallel_loop(0, CHUNK, step=LANES, unroll=16)
            def _(i):
                s = pl.ds(i, LANES)
                t = tv[b][s]
                g16 = 1.0 / (1.0 + jnp.exp(-t))
                gv[b][s] = g16
                ov[b][s] = yv[b][s] + g16 * (xv[b][s] - yv[b][s])

            @pl.when(c + DEPTH < NCHUNK)
            def _():
                start_in(c + DEPTH, b)

            start_out(c, b)

        return carry

    lax.fori_loop(0, NCHUNK // DEPTH, round_body, 0)
    for b in range(DEPTH):
        drain_out(b)


def kernel(X, Y, gate_theta):
    x = X.reshape(E)
    y = Y.reshape(E)
    t = gate_theta.reshape(E)
    out, gate = _gate_fuse(x, y, t)
    return out.reshape(X.shape), gate.reshape(X.shape)


# speedup vs baseline: 1.0118x; 1.0037x over previous
import functools

import jax
import jax.numpy as jnp
from jax import lax
from jax.experimental import pallas as pl
from jax.experimental.pallas import tpu as pltpu
from jax.experimental.pallas import tpu_sc as plsc

ENTITY_NUM = 100000
HIDDEN_DIM = 128
E = ENTITY_NUM * HIDDEN_DIM

NC = 2
NS = 16
NW = NC * NS
LANES = 16

PER_W = E // NW
DEPTH = 2
CHUNK = 10000
NCHUNK = PER_W // CHUNK

_mesh = plsc.VectorSubcoreMesh(core_axis_name="c", subcore_axis_name="s")


@functools.partial(
    pl.kernel,
    mesh=_mesh,
    out_type=[
        jax.ShapeDtypeStruct((E,), jnp.float32),
        jax.ShapeDtypeStruct((E,), jnp.float32),
    ],
    scratch_types=(
        [pltpu.VMEM((CHUNK,), jnp.float32)] * (5 * DEPTH)
        + [pltpu.SemaphoreType.DMA] * (2 * DEPTH)
    ),
)
def _gate_fuse(x_hbm, y_hbm, t_hbm, out_hbm, gate_hbm, *scratch):
    bufs = scratch[: 5 * DEPTH]
    sems = scratch[5 * DEPTH:]
    xv = bufs[0:DEPTH]
    yv = bufs[DEPTH:2 * DEPTH]
    tv = bufs[2 * DEPTH:3 * DEPTH]
    ov = bufs[3 * DEPTH:4 * DEPTH]
    gv = bufs[4 * DEPTH:5 * DEPTH]
    sem_in = sems[0:DEPTH]
    sem_out = sems[DEPTH:2 * DEPTH]

    wid = lax.axis_index("s") * NC + lax.axis_index("c")
    base = wid * CHUNK

    def start_in(c, b):
        off = base + c * (NW * CHUNK)
        pltpu.async_copy(x_hbm.at[pl.ds(off, CHUNK)], xv[b], sem_in[b])
        pltpu.async_copy(y_hbm.at[pl.ds(off, CHUNK)], yv[b], sem_in[b])
        pltpu.async_copy(t_hbm.at[pl.ds(off, CHUNK)], tv[b], sem_in[b])

    def drain_in(b):
        for dst in (xv[b], yv[b], tv[b]):
            pltpu.make_async_copy(x_hbm.at[pl.ds(0, CHUNK)], dst, sem_in[b]).wait()

    def start_out(c, b):
        off = base + c * (NW * CHUNK)
        pltpu.async_copy(ov[b], out_hbm.at[pl.ds(off, CHUNK)], sem_out[b])
        pltpu.async_copy(gv[b], gate_hbm.at[pl.ds(off, CHUNK)], sem_out[b])

    def drain_out(b):
        pltpu.make_async_copy(ov[b], out_hbm.at[pl.ds(0, CHUNK)], sem_out[b]).wait()
        pltpu.make_async_copy(gv[b], gate_hbm.at[pl.ds(0, CHUNK)], sem_out[b]).wait()

    for b in range(DEPTH):
        start_in(b, b)

    def round_body(g, carry):
        for b in range(DEPTH):
            c = DEPTH * g + b
            drain_in(b)

            @pl.when(g > 0)
            def _():
                drain_out(b)

            @plsc.parallel_loop(0, CHUNK, step=LANES, unroll=16)
            def _(i):
                s = pl.ds(i, LANES)
                t = tv[b][s]
                g16 = 1.0 / (1.0 + jnp.exp(-t))
                gv[b][s] = g16
                ov[b][s] = yv[b][s] + g16 * (xv[b][s] - yv[b][s])

            @pl.when(c + DEPTH < NCHUNK)
            def _():
                start_in(c + DEPTH, b)

            start_out(c, b)

        return carry

    lax.fori_loop(0, NCHUNK // DEPTH, round_body, 0)
    for b in range(DEPTH):
        drain_out(b)


def kernel(X, Y, gate_theta):
    x = X.reshape(E)
    y = Y.reshape(E)
    t = gate_theta.reshape(E)
    out, gate = _gate_fuse(x, y, t)
    return out.reshape(X.shape), gate.reshape(X.shape)
